# baseline (speedup 1.0000x reference)
import jax
import jax.numpy as jnp
from jax.experimental import pallas as pl
from jax.experimental.pallas import tpu as pltpu

_EPS = 1e-8
_G = 8


def _body(x_ref, h0_ref, h1_ref, h2_ref, h3_ref, mem_ref, wf_ref,
          w0_ref, w1_ref, w2_ref, w3_ref, out_ref, *, tseq, rows):
    grp = _G * tseq
    n_grp = rows // grp

    x = x_ref[...]
    e = jnp.dot(x, wf_ref[...], preferred_element_type=jnp.float32)
    e = e - jnp.max(e, axis=-1, keepdims=True)
    p = jnp.exp(e)
    p = p / jnp.sum(p, axis=-1, keepdims=True)
    mems = jnp.dot(p, mem_ref[...], preferred_element_type=jnp.float32)
    nm = jnp.maximum(jnp.sqrt(jnp.sum(mems * mems, axis=-1, keepdims=True)),
                     _EPS)

    r = jax.lax.broadcasted_iota(jnp.int32, (grp, grp), 0) // tseq
    c = jax.lax.broadcasted_iota(jnp.int32, (grp, grp), 1) // tseq
    blockmask = r == c

    for i, (h_ref, w_ref) in enumerate(
            ((h0_ref, w0_ref), (h1_ref, w1_ref),
             (h2_ref, w2_ref), (h3_ref, w3_ref))):
        h = h_ref[...]
        qkv = jnp.dot(h, w_ref[...], preferred_element_type=jnp.float32)
        q = qkv[:, 0:32]
        k = qkv[:, 32:64]
        v = qkv[:, 64:96]
        outs = []
        for g in range(n_grp):
            sl = slice(g * grp, (g + 1) * grp)
            qg = q[sl]
            kg = k[sl]
            vg = v[sl]
            eg = jax.lax.dot_general(qg, kg, (((1,), (1,)), ((), ())),
                                     preferred_element_type=jnp.float32)
            eg = jnp.where(blockmask, eg, -1e30)
            eg = eg - jnp.max(eg, axis=-1, keepdims=True)
            pg = jnp.exp(eg)
            pg = pg / jnp.sum(pg, axis=-1, keepdims=True)
            outs.append(jnp.dot(pg, vg, preferred_element_type=jnp.float32))
        att = jnp.concatenate(outs, axis=0)
        na = jnp.maximum(jnp.sqrt(jnp.sum(att * att, axis=-1, keepdims=True)),
                         _EPS)
        cos = jnp.sum(mems * att, axis=-1, keepdims=True) / (nm * na)
        out_ref[:, i:i + 1] = cos


def kernel(input, hidden_0, hidden_1, hidden_2, hidden_3, memory, input_query,
           hid_query_0, hid_query_1, hid_query_2, hid_query_3,
           key_0, key_1, key_2, key_3,
           value_0, value_1, value_2, value_3):
    B, N, T, IN_DIM = input.shape
    HID = hidden_0.shape[-1]
    MEM, MH = memory.shape
    BN = B * N
    total = BN * T

    tile_p = 64
    rows = tile_p * T
    steps = BN // tile_p

    x = input.reshape(total, IN_DIM)
    hs = [h.reshape(total, HID)
          for h in (hidden_0, hidden_1, hidden_2, hidden_3)]
    wf = jnp.dot(input_query, memory.T)
    ws = [jnp.concatenate([hq, kk, vv], axis=1)
          for hq, kk, vv in ((hid_query_0, key_0, value_0),
                             (hid_query_1, key_1, value_1),
                             (hid_query_2, key_2, value_2),
                             (hid_query_3, key_3, value_3))]

    row_spec = lambda width: pl.BlockSpec((rows, width), lambda i: (i, 0))
    full_spec = lambda a: pl.BlockSpec(a.shape, lambda i: (0, 0))

    import functools
    body = functools.partial(_body, tseq=T, rows=rows)

    scores = pl.pallas_call(
        body,
        grid=(steps,),
        in_specs=[row_spec(IN_DIM)] + [row_spec(HID)] * 4
                 + [full_spec(memory), full_spec(wf)]
                 + [full_spec(w) for w in ws],
        out_specs=pl.BlockSpec((rows, 4), lambda i: (i, 0)),
        out_shape=jax.ShapeDtypeStruct((total, 4), jnp.float32),
        compiler_params=pltpu.CompilerParams(
            dimension_semantics=("parallel",)),
    )(x, *hs, memory, wf, *ws)

    return scores.reshape(B, N, T, 1, 4)


# speedup vs baseline: 2.0165x; 2.0165x over previous
import functools
import numpy as np
import jax
import jax.numpy as jnp
from jax.experimental import pallas as pl
from jax.experimental.pallas import tpu as pltpu

_EPS = 1e-8
_G = 8


def _body(x_ref, h0_ref, h1_ref, h2_ref, h3_ref, mem_ref, wf_ref, bias_ref,
          w0_ref, w1_ref, w2_ref, w3_ref, out_ref, *, tseq, rows):
    grp = _G * tseq
    n_grp = rows // grp
    bias = bias_ref[...]

    x = x_ref[...]
    e = jnp.dot(x, wf_ref[...], preferred_element_type=jnp.float32)
    p = jnp.exp(e)
    mems = jnp.dot(p, mem_ref[...], preferred_element_type=jnp.float32)
    nm = jnp.maximum(jnp.sqrt(jnp.sum(mems * mems, axis=-1, keepdims=True)),
                     _EPS)

    for i, (h_ref, w_ref) in enumerate(
            ((h0_ref, w0_ref), (h1_ref, w1_ref),
             (h2_ref, w2_ref), (h3_ref, w3_ref))):
        h = h_ref[...]
        qkv = jnp.dot(h, w_ref[...], preferred_element_type=jnp.float32)
        q = qkv[:, 0:32]
        k = qkv[:, 32:64]
        v = qkv[:, 64:96]
        egs = []
        for g in range(n_grp):
            sl = slice(g * grp, (g + 1) * grp)
            egs.append(bias + jax.lax.dot_general(
                q[sl], k[sl], (((1,), (1,)), ((), ())),
                preferred_element_type=jnp.float32))
        en = jnp.concatenate(egs, axis=0)
        en = en - jnp.max(en, axis=-1, keepdims=True)
        pr = jnp.exp(en)
        outs = []
        for g in range(n_grp):
            sl = slice(g * grp, (g + 1) * grp)
            outs.append(jnp.dot(pr[sl], v[sl],
                                preferred_element_type=jnp.float32))
        att = jnp.concatenate(outs, axis=0)
        na = jnp.maximum(jnp.sqrt(jnp.sum(att * att, axis=-1, keepdims=True)),
                         _EPS)
        cos = jnp.sum(mems * att, axis=-1, keepdims=True) / (nm * na)
        out_ref[:, i:i + 1] = cos


def kernel(input, hidden_0, hidden_1, hidden_2, hidden_3, memory, input_query,
           hid_query_0, hid_query_1, hid_query_2, hid_query_3,
           key_0, key_1, key_2, key_3,
           value_0, value_1, value_2, value_3):
    B, N, T, IN_DIM = input.shape
    HID = hidden_0.shape[-1]
    BN = B * N
    total = BN * T

    tile_p = 64
    rows = tile_p * T
    steps = BN // tile_p
    grp = _G * T

    x = input.reshape(total, IN_DIM)
    hs = [h.reshape(total, HID)
          for h in (hidden_0, hidden_1, hidden_2, hidden_3)]
    wf = jnp.dot(input_query, memory.T)
    ws = [jnp.concatenate([hq, kk, vv], axis=1)
          for hq, kk, vv in ((hid_query_0, key_0, value_0),
                             (hid_query_1, key_1, value_1),
                             (hid_query_2, key_2, value_2),
                             (hid_query_3, key_3, value_3))]
    rr = np.arange(grp) // T
    bias = jnp.asarray(
        np.where(rr[:, None] == rr[None, :], 0.0, -1e30), jnp.float32)

    row_spec = lambda width: pl.BlockSpec((rows, width), lambda i: (i, 0))
    full_spec = lambda a: pl.BlockSpec(a.shape, lambda i: (0, 0))

    body = functools.partial(_body, tseq=T, rows=rows)

    scores = pl.pallas_call(
        body,
        grid=(steps,),
        in_specs=[row_spec(IN_DIM)] + [row_spec(HID)] * 4
                 + [full_spec(memory), full_spec(wf), full_spec(bias)]
                 + [full_spec(w) for w in ws],
        out_specs=pl.BlockSpec((rows, 4), lambda i: (i, 0)),
        out_shape=jax.ShapeDtypeStruct((total, 4), jnp.float32),
        compiler_params=pltpu.CompilerParams(
            dimension_semantics=("parallel",)),
    )(x, *hs, memory, wf, bias, *ws)

    return scores.reshape(B, N, T, 1, 4)


# speedup vs baseline: 2.5419x; 1.2606x over previous
import functools
import numpy as np
import jax
import jax.numpy as jnp
from jax.experimental import pallas as pl
from jax.experimental.pallas import tpu as pltpu

_EPS2 = 1e-30
_G = 8


def _body(x_ref, h0_ref, h1_ref, h2_ref, h3_ref, mem_ref, wf_ref, bias_ref,
          eye4_ref, s8_ref, ones_ref, out_ref, *, tseq, rows, w_refs):
    grp = _G * tseq
    n_grp = rows // grp
    bias = bias_ref[...]

    x = x_ref[...]
    e = jnp.dot(x, wf_ref[...], preferred_element_type=jnp.float32)
    p = jnp.exp(e)
    mems = jnp.dot(p, mem_ref[...], preferred_element_type=jnp.float32)
    n2m = jnp.dot(mems * mems, ones_ref[...],
                  preferred_element_type=jnp.float32)
    inv_m = jax.lax.rsqrt(jnp.maximum(n2m, _EPS2))

    qkvs = [jnp.dot(h_ref[...], w_ref[...],
                    preferred_element_type=jnp.float32)
            for h_ref, w_ref in zip((h0_ref, h1_ref, h2_ref, h3_ref), w_refs)]

    outs = []
    for qkv in qkvs:
        egs = []
        for g in range(n_grp):
            sl = slice(g * grp, (g + 1) * grp)
            egs.append(bias + jax.lax.dot_general(
                qkv[sl, 0:32], qkv[sl, 32:64], (((1,), (1,)), ((), ())),
                preferred_element_type=jnp.float32))
        en = jnp.concatenate(egs, axis=0)
        en = en - jnp.max(en, axis=-1, keepdims=True)
        pr = jnp.exp(en)
        outs.append([jnp.dot(pr[g * grp:(g + 1) * grp],
                             qkv[g * grp:(g + 1) * grp, 64:96],
                             preferred_element_type=jnp.float32)
                     for g in range(n_grp)])
    att = jnp.concatenate(
        [jnp.concatenate([outs[i][g] for i in range(4)], axis=1)
         for g in range(n_grp)], axis=0)
    m4 = jnp.dot(mems, eye4_ref[...],
                 preferred_element_type=jnp.float32)
    packed = jnp.concatenate([att * att, att * m4], axis=1)
    prods = jnp.dot(packed, s8_ref[...],
                    preferred_element_type=jnp.float32)
    inv_a = jax.lax.rsqrt(jnp.maximum(prods[:, 0:4], _EPS2))
    out_ref[...] = prods[:, 4:8] * inv_a * inv_m


def kernel(input, hidden_0, hidden_1, hidden_2, hidden_3, memory, input_query,
           hid_query_0, hid_query_1, hid_query_2, hid_query_3,
           key_0, key_1, key_2, key_3,
           value_0, value_1, value_2, value_3):
    B, N, T, IN_DIM = input.shape
    HID = hidden_0.shape[-1]
    MH = memory.shape[1]
    BN = B * N
    total = BN * T

    tile_p = 64
    rows = tile_p * T
    steps = BN // tile_p
    grp = _G * T

    x = input.reshape(total, IN_DIM)
    hs = [h.reshape(total, HID)
          for h in (hidden_0, hidden_1, hidden_2, hidden_3)]
    wf = jnp.dot(input_query, memory.T)
    ws = [jnp.concatenate([hq, kk, vv], axis=1)
          for hq, kk, vv in ((hid_query_0, key_0, value_0),
                             (hid_query_1, key_1, value_1),
                             (hid_query_2, key_2, value_2),
                             (hid_query_3, key_3, value_3))]

    rr = np.arange(grp) // T
    bias = jnp.asarray(
        np.where(rr[:, None] == rr[None, :], 0.0, -1e30), jnp.float32)
    eye4 = jnp.asarray(np.tile(np.eye(MH, dtype=np.float32), (1, 4)))
    s8 = np.zeros((8 * MH, 8), np.float32)
    for j in range(8):
        s8[j * MH:(j + 1) * MH, j] = 1.0
    s8 = jnp.asarray(s8)
    ones = jnp.ones((MH, 1), jnp.float32)

    row_spec = lambda width: pl.BlockSpec((rows, width), lambda i: (i, 0))
    full_spec = lambda a: pl.BlockSpec(a.shape, lambda i: (0, 0))

    def body_fn(x_ref, h0, h1, h2, h3, mem_ref, wf_ref, bias_ref,
                eye4_ref, s8_ref, ones_ref, w0, w1, w2, w3, out_ref):
        _body(x_ref, h0, h1, h2, h3, mem_ref, wf_ref, bias_ref,
              eye4_ref, s8_ref, ones_ref, out_ref,
              tseq=T, rows=rows, w_refs=(w0, w1, w2, w3))

    scores = pl.pallas_call(
        body_fn,
        grid=(steps,),
        in_specs=[row_spec(IN_DIM)] + [row_spec(HID)] * 4
                 + [full_spec(a) for a in (memory, wf, bias, eye4, s8, ones)]
                 + [full_spec(w) for w in ws],
        out_specs=pl.BlockSpec((rows, 4), lambda i: (i, 0)),
        out_shape=jax.ShapeDtypeStruct((total, 4), jnp.float32),
        compiler_params=pltpu.CompilerParams(
            dimension_semantics=("parallel",)),
    )(x, *hs, memory, wf, bias, eye4, s8, ones, *ws)

    return scores.reshape(B, N, T, 1, 4)


# speedup vs baseline: 2.5496x; 1.0030x over previous
import functools
import numpy as np
import jax
import jax.numpy as jnp
from jax.experimental import pallas as pl
from jax.experimental.pallas import tpu as pltpu

_EPS2 = 1e-30
_G = 8


def _body(x_ref, h0_ref, h1_ref, h2_ref, h3_ref, mem_ref, wf_ref, bias_ref,
          eye4_ref, s8_ref, ones_ref, out_ref, *, tseq, rows, w_refs):
    grp = _G * tseq
    n_grp = rows // grp
    bias = bias_ref[...]

    x = x_ref[...]
    e = jnp.dot(x, wf_ref[...], preferred_element_type=jnp.float32)
    p = jnp.exp(e)
    mems = jnp.dot(p, mem_ref[...], preferred_element_type=jnp.float32)
    n2m = jnp.dot(mems * mems, ones_ref[...],
                  preferred_element_type=jnp.float32)
    inv_m = jax.lax.rsqrt(jnp.maximum(n2m, _EPS2))

    qkvs = [jnp.dot(h_ref[...], w_ref[...],
                    preferred_element_type=jnp.float32)
            for h_ref, w_ref in zip((h0_ref, h1_ref, h2_ref, h3_ref), w_refs)]

    outs = []
    for qkv in qkvs:
        egs = []
        for g in range(n_grp):
            sl = slice(g * grp, (g + 1) * grp)
            egs.append(bias + jax.lax.dot_general(
                qkv[sl, 0:32], qkv[sl, 32:64], (((1,), (1,)), ((), ())),
                preferred_element_type=jnp.float32))
        en = jnp.concatenate(egs, axis=0)
        en = en - jnp.max(en, axis=-1, keepdims=True)
        pr = jnp.exp(en)
        outs.append([jnp.dot(pr[g * grp:(g + 1) * grp],
                             qkv[g * grp:(g + 1) * grp, 64:96],
                             preferred_element_type=jnp.float32)
                     for g in range(n_grp)])
    att = jnp.concatenate(
        [jnp.concatenate([outs[i][g] for i in range(4)], axis=1)
         for g in range(n_grp)], axis=0)
    m4 = jnp.dot(mems, eye4_ref[...],
                 preferred_element_type=jnp.float32)
    packed = jnp.concatenate([att * att, att * m4], axis=1)
    prods = jnp.dot(packed, s8_ref[...],
                    preferred_element_type=jnp.float32)
    inv_a = jax.lax.rsqrt(jnp.maximum(prods[:, 0:4], _EPS2))
    out_ref[...] = prods[:, 4:8] * inv_a * inv_m


def kernel(input, hidden_0, hidden_1, hidden_2, hidden_3, memory, input_query,
           hid_query_0, hid_query_1, hid_query_2, hid_query_3,
           key_0, key_1, key_2, key_3,
           value_0, value_1, value_2, value_3):
    B, N, T, IN_DIM = input.shape
    HID = hidden_0.shape[-1]
    MH = memory.shape[1]
    BN = B * N
    total = BN * T

    tile_p = 160
    rows = tile_p * T
    steps = BN // tile_p
    grp = _G * T

    x = input.reshape(total, IN_DIM)
    hs = [h.reshape(total, HID)
          for h in (hidden_0, hidden_1, hidden_2, hidden_3)]
    wf = jnp.dot(input_query, memory.T)
    ws = [jnp.concatenate([hq, kk, vv], axis=1)
          for hq, kk, vv in ((hid_query_0, key_0, value_0),
                             (hid_query_1, key_1, value_1),
                             (hid_query_2, key_2, value_2),
                             (hid_query_3, key_3, value_3))]

    rr = np.arange(grp) // T
    bias = jnp.asarray(
        np.where(rr[:, None] == rr[None, :], 0.0, -1e30), jnp.float32)
    eye4 = jnp.asarray(np.tile(np.eye(MH, dtype=np.float32), (1, 4)))
    s8 = np.zeros((8 * MH, 8), np.float32)
    for j in range(8):
        s8[j * MH:(j + 1) * MH, j] = 1.0
    s8 = jnp.asarray(s8)
    ones = jnp.ones((MH, 1), jnp.float32)

    row_spec = lambda width: pl.BlockSpec((rows, width), lambda i: (i, 0))
    full_spec = lambda a: pl.BlockSpec(a.shape, lambda i: (0, 0))

    def body_fn(x_ref, h0, h1, h2, h3, mem_ref, wf_ref, bias_ref,
                eye4_ref, s8_ref, ones_ref, w0, w1, w2, w3, out_ref):
        _body(x_ref, h0, h1, h2, h3, mem_ref, wf_ref, bias_ref,
              eye4_ref, s8_ref, ones_ref, out_ref,
              tseq=T, rows=rows, w_refs=(w0, w1, w2, w3))

    scores = pl.pallas_call(
        body_fn,
        grid=(steps,),
        in_specs=[row_spec(IN_DIM)] + [row_spec(HID)] * 4
                 + [full_spec(a) for a in (memory, wf, bias, eye4, s8, ones)]
                 + [full_spec(w) for w in ws],
        out_specs=pl.BlockSpec((rows, 4), lambda i: (i, 0)),
        out_shape=jax.ShapeDtypeStruct((total, 4), jnp.float32),
        compiler_params=pltpu.CompilerParams(
            dimension_semantics=("parallel",)),
    )(x, *hs, memory, wf, bias, eye4, s8, ones, *ws)

    return scores.reshape(B, N, T, 1, 4)


# speedup vs baseline: 3.5309x; 1.3849x over previous
import numpy as np
import jax
import jax.numpy as jnp
from jax.experimental import pallas as pl
from jax.experimental.pallas import tpu as pltpu

_EPS2 = 1e-30
_G = 8


def _body(x_ref, h0_ref, h1_ref, h2_ref, h3_ref, mem_ref, wf_ref, bias_ref,
          eye4_ref, s8_ref, ones_ref, out_ref, *, tseq, rows, w_refs):
    grp = _G * tseq
    n_grp = rows // grp
    hid = h0_ref.shape[-1]
    bias = bias_ref[...]

    x = x_ref[...].reshape(rows, x_ref.shape[-1])
    e = jnp.dot(x, wf_ref[...], preferred_element_type=jnp.float32)
    p = jnp.exp(e)
    mems = jnp.dot(p, mem_ref[...], preferred_element_type=jnp.float32)
    n2m = jnp.dot(mems * mems, ones_ref[...],
                  preferred_element_type=jnp.float32)
    inv_m = jax.lax.rsqrt(jnp.maximum(n2m, _EPS2))

    qkvs = [jnp.dot(h_ref[...].reshape(rows, hid), w_ref[...],
                    preferred_element_type=jnp.float32)
            for h_ref, w_ref in zip((h0_ref, h1_ref, h2_ref, h3_ref), w_refs)]

    outs = []
    for qkv in qkvs:
        egs = []
        for g in range(n_grp):
            sl = slice(g * grp, (g + 1) * grp)
            egs.append(bias + jax.lax.dot_general(
                qkv[sl, 0:32], qkv[sl, 32:64], (((1,), (1,)), ((), ())),
                preferred_element_type=jnp.float32))
        en = jnp.concatenate(egs, axis=0)
        en = en - jnp.max(en, axis=-1, keepdims=True)
        pr = jnp.exp(en)
        outs.append([jnp.dot(pr[g * grp:(g + 1) * grp],
                             qkv[g * grp:(g + 1) * grp, 64:96],
                             preferred_element_type=jnp.float32)
                     for g in range(n_grp)])
    att = jnp.concatenate(
        [jnp.concatenate([outs[i][g] for i in range(4)], axis=1)
         for g in range(n_grp)], axis=0)
    m4 = jnp.dot(mems, eye4_ref[...],
                 preferred_element_type=jnp.float32)
    packed = jnp.concatenate([att * att, att * m4], axis=1)
    prods = jnp.dot(packed, s8_ref[...],
                    preferred_element_type=jnp.float32)
    inv_a = jax.lax.rsqrt(jnp.maximum(prods[:, 0:4], _EPS2))
    out_ref[...] = prods[:, 4:8] * inv_a * inv_m


def kernel(input, hidden_0, hidden_1, hidden_2, hidden_3, memory, input_query,
           hid_query_0, hid_query_1, hid_query_2, hid_query_3,
           key_0, key_1, key_2, key_3,
           value_0, value_1, value_2, value_3):
    B, N, T, IN_DIM = input.shape
    HID = hidden_0.shape[-1]
    MH = memory.shape[1]
    BN = B * N
    total = BN * T

    tile_p = 200
    rows = tile_p * T
    steps = BN // tile_p
    grp = _G * T

    x = input.reshape(BN, T, IN_DIM)
    hs = [h.reshape(BN, T, HID)
          for h in (hidden_0, hidden_1, hidden_2, hidden_3)]
    wf = jnp.dot(input_query, memory.T)
    ws = [jnp.concatenate([hq, kk, vv], axis=1)
          for hq, kk, vv in ((hid_query_0, key_0, value_0),
                             (hid_query_1, key_1, value_1),
                             (hid_query_2, key_2, value_2),
                             (hid_query_3, key_3, value_3))]

    rr = np.arange(grp) // T
    bias = jnp.asarray(
        np.where(rr[:, None] == rr[None, :], 0.0, -1e30), jnp.float32)
    eye4 = jnp.asarray(np.tile(np.eye(MH, dtype=np.float32), (1, 4)))
    s8 = np.zeros((8 * MH, 8), np.float32)
    for j in range(8):
        s8[j * MH:(j + 1) * MH, j] = 1.0
    s8 = jnp.asarray(s8)
    ones = jnp.ones((MH, 1), jnp.float32)

    row_spec = lambda width: pl.BlockSpec((tile_p, T, width),
                                          lambda i: (i, 0, 0))
    full_spec = lambda a: pl.BlockSpec(a.shape, lambda i: (0,) * a.ndim)

    def body_fn(x_ref, h0, h1, h2, h3, mem_ref, wf_ref, bias_ref,
                eye4_ref, s8_ref, ones_ref, w0, w1, w2, w3, out_ref):
        _body(x_ref, h0, h1, h2, h3, mem_ref, wf_ref, bias_ref,
              eye4_ref, s8_ref, ones_ref, out_ref,
              tseq=T, rows=rows, w_refs=(w0, w1, w2, w3))

    scores = pl.pallas_call(
        body_fn,
        grid=(steps,),
        in_specs=[row_spec(IN_DIM)] + [row_spec(HID)] * 4
                 + [full_spec(a) for a in (memory, wf, bias, eye4, s8, ones)]
                 + [full_spec(w) for w in ws],
        out_specs=pl.BlockSpec((rows, 4), lambda i: (i, 0)),
        out_shape=jax.ShapeDtypeStruct((total, 4), jnp.float32),
        compiler_params=pltpu.CompilerParams(
            dimension_semantics=("parallel",)),
    )(x, *hs, memory, wf, bias, eye4, s8, ones, *ws)

    return scores.reshape(B, N, T, 1, 4)
